# baseline (speedup 1.0000x reference)
import functools

import jax
import jax.numpy as jnp
from jax import lax
from jax.experimental import pallas as pl
from jax.experimental.pallas import tpu as pltpu
from jax.experimental.pallas import tpu_sc as plsc

_N_ITER = 40
_NC = 2
_NS = 16
_RPT = 320
_HALF = _NS * _RPT
_NP = _NC * _HALF
_CH = 128
_D = 128
_W = 5
_EPI = ((0, 128), (128, 128), (256, 64))


def _sc_prop_body(a_hbm, b_hbm, a2_hbm, b2_hbm, cols_hbm, rloc_hbm, cnts_hbm,
                  zb_hbm, flags_hbm, vout_hbm,
                  acc_sh, gbuf, av, bv, g3, g4, idxcs, idxrs, cntv, fbuf,
                  fbuf2, sem_i, sem_g, sem_s):
    c = lax.axis_index("c")
    s = lax.axis_index("s")
    oc = 1 - c
    base_l = s * _RPT
    base_g = c * _HALF + base_l
    bufs = (gbuf, av, bv, g3, g4)

    def fence(k):
        @pl.when(s == 0)
        def _fence():
            fbuf[...] = jnp.broadcast_to(k, (16,)).astype(jnp.int32)
            pltpu.sync_copy(fbuf, flags_hbm.at[c])

            def _poll(i, done):
                @pl.when(jnp.logical_not(done))
                def _rd():
                    pltpu.sync_copy(flags_hbm.at[oc], fbuf2)

                now = jnp.logical_or(done, fbuf2[...][0] >= k)

                @pl.when(jnp.logical_not(now))
                def _nap():
                    pl.delay(500)

                return now

            lax.fori_loop(0, 8192, _poll, jnp.bool_(False))

    pltpu.sync_copy(cnts_hbm, cntv)
    cnt_vec = cntv[...]
    nch = jnp.where(c == 0, cnt_vec[0], cnt_vec[1])
    my_n = jnp.maximum(0, (nch - s + _NS - 1) // _NS)
    n_waves = (my_n + _W - 1) // _W

    def zero_acc_slice():
        pltpu.sync_copy(zb_hbm, gbuf)
        for blk, bsz in _EPI:
            pltpu.sync_copy(gbuf.at[pl.ds(0, bsz)],
                            acc_sh.at[pl.ds(base_l + blk, bsz)])

    for blk, bsz in _EPI:
        pltpu.sync_copy(a_hbm.at[pl.ds(base_g + blk, bsz)],
                        av.at[pl.ds(0, bsz)])
        pltpu.sync_copy(av.at[pl.ds(0, bsz)],
                        vout_hbm.at[pl.ds(base_g + blk, bsz)])
    zero_acc_slice()
    plsc.subcore_barrier()
    fence(jnp.int32(1))
    plsc.subcore_barrier()

    def scatter_phase():
        def wave_body(w, carry):
            ts = [w * _W + b for b in range(_W)]
            valid = [t < my_n for t in ts]
            ebases = [(t * _NS + s) * _CH for t in ts]
            for b in range(_W):
                @pl.when(valid[b])
                def _issue_idx(b=b):
                    pltpu.async_copy(cols_hbm.at[c, pl.ds(ebases[b], _CH)],
                                     idxcs.at[b], sem_i)
                    pltpu.async_copy(rloc_hbm.at[c, pl.ds(ebases[b], _CH)],
                                     idxrs.at[b], sem_i)
            for b in range(_W):
                @pl.when(valid[b])
                def _wait_idx(b=b):
                    pltpu.make_async_copy(
                        cols_hbm.at[c, pl.ds(ebases[b], _CH)],
                        idxcs.at[b], sem_i).wait()
                    pltpu.make_async_copy(
                        rloc_hbm.at[c, pl.ds(ebases[b], _CH)],
                        idxrs.at[b], sem_i).wait()
            for b in range(_W):
                @pl.when(valid[b])
                def _issue_gather(b=b):
                    pltpu.async_copy(
                        vout_hbm.at[plsc.Indices(idxcs.at[b],
                                                 ignored_value=-1)],
                        bufs[b], sem_g)
            for b in range(_W):
                @pl.when(valid[b])
                def _gather_scatter(b=b):
                    pltpu.make_async_copy(
                        vout_hbm.at[plsc.Indices(idxcs.at[b],
                                                 ignored_value=-1)],
                        bufs[b], sem_g).wait()
                    pltpu.async_copy(
                        bufs[b],
                        acc_sh.at[plsc.Indices(idxrs.at[b], ignored_value=-1)],
                        sem_s, add=True)
            for b in range(_W):
                @pl.when(valid[b])
                def _wait_scatter(b=b):
                    pltpu.make_async_copy(
                        bufs[b],
                        acc_sh.at[plsc.Indices(idxrs.at[b], ignored_value=-1)],
                        sem_s).wait()
            return carry

        lax.fori_loop(0, n_waves, wave_body, 0)

    def axpy_blocks(a_ref, b_ref):
        for blk, bsz in _EPI:
            pltpu.sync_copy(acc_sh.at[pl.ds(base_l + blk, bsz)],
                            gbuf.at[pl.ds(0, bsz)])
            pltpu.sync_copy(a_ref.at[pl.ds(base_g + blk, bsz)],
                            av.at[pl.ds(0, bsz)])
            pltpu.sync_copy(b_ref.at[pl.ds(base_g + blk, bsz)],
                            bv.at[pl.ds(0, bsz)])

            def row_body(r, carry):
                for cc in range(_D // 16):
                    sl = pl.ds(cc * 16, 16)
                    av[r, sl] = av[r, sl] + bv[r, sl] * gbuf[r, sl]
                return carry

            lax.fori_loop(0, bsz, row_body, 0)
            pltpu.sync_copy(av.at[pl.ds(0, bsz)],
                            vout_hbm.at[pl.ds(base_g + blk, bsz)])

    def iter_body(t, carry):
        scatter_phase()
        plsc.subcore_barrier()
        fence(2 * t + 2)
        plsc.subcore_barrier()

        @pl.when(t < _N_ITER - 1)
        def _epilogue_iter():
            axpy_blocks(a_hbm, b_hbm)
            zero_acc_slice()
            plsc.subcore_barrier()
            fence(2 * t + 3)
            plsc.subcore_barrier()

        @pl.when(t == _N_ITER - 1)
        def _epilogue_last():
            axpy_blocks(a2_hbm, b2_hbm)

        return carry

    lax.fori_loop(0, _N_ITER, iter_body, 0)


_sc_prop = functools.partial(
    pl.kernel,
    out_type=jax.ShapeDtypeStruct((_NP, _D), jnp.float32),
    mesh=plsc.VectorSubcoreMesh(core_axis_name="c", subcore_axis_name="s"),
    scratch_types=[
        pltpu.VMEM_SHARED((_HALF, _D), jnp.float32),
        pltpu.VMEM((_CH, _D), jnp.float32),
        pltpu.VMEM((_CH, _D), jnp.float32),
        pltpu.VMEM((_CH, _D), jnp.float32),
        pltpu.VMEM((_CH, _D), jnp.float32),
        pltpu.VMEM((_CH, _D), jnp.float32),
        pltpu.VMEM((_W, _CH), jnp.int32),
        pltpu.VMEM((_W, _CH), jnp.int32),
        pltpu.VMEM((16,), jnp.int32),
        pltpu.VMEM((16,), jnp.int32),
        pltpu.VMEM((16,), jnp.int32),
        pltpu.SemaphoreType.DMA,
        pltpu.SemaphoreType.DMA,
        pltpu.SemaphoreType.DMA,
    ],
)(_sc_prop_body)


def kernel(x, edge_index, mask):
    n, d = x.shape
    e = edge_index.shape[1]
    row = edge_index[0].astype(jnp.int32)
    col = edge_index[1].astype(jnp.int32)
    maskf = mask.astype(jnp.float32)

    ones = jnp.ones((e,), jnp.float32)
    deg = jnp.zeros((n,), jnp.float32).at[col].add(ones)
    dis = jnp.where(deg > 0, jax.lax.rsqrt(deg), 0.0)

    in0 = row < _HALF
    pos0 = jnp.cumsum(in0.astype(jnp.int32)) - 1
    n0 = pos0[-1] + 1
    pos1 = jnp.cumsum(1 - in0.astype(jnp.int32)) - 1
    pos = jnp.where(in0, pos0, e + pos1)
    cols2 = jnp.full((2 * e,), -1, jnp.int32).at[pos].set(col)
    rloc2 = jnp.full((2 * e,), -1, jnp.int32).at[pos].set(
        jnp.where(in0, row, row - _HALF))
    cols_p = cols2.reshape(2, e)
    rloc_p = rloc2.reshape(2, e)
    n1 = e - n0
    cnts = jnp.zeros((16,), jnp.int32).at[0].set(
        (n0 + _CH - 1) // _CH).at[1].set((n1 + _CH - 1) // _CH)

    xm = jnp.where(mask[:, None], x, 0.0).astype(jnp.float32)
    xm_p = jnp.zeros((_NP, d), jnp.float32).at[:n].set(xm)
    dis_p = jnp.zeros((_NP,), jnp.float32).at[:n].set(dis)
    nm_p = jnp.zeros((_NP,), jnp.float32).at[:n].set(1.0 - maskf)

    a_iter = xm_p * dis_p[:, None]
    b_iter = jnp.broadcast_to((nm_p * dis_p * dis_p)[:, None], (_NP, d))
    a_last = xm_p
    b_last = jnp.broadcast_to((nm_p * dis_p)[:, None], (_NP, d))
    zblk = jnp.zeros((_CH, _D), jnp.float32)

    flags = jax.new_ref(jnp.zeros((_NC, 16), jnp.int32))
    out = _sc_prop(a_iter, b_iter, a_last, b_last,
                   cols_p, rloc_p, cnts, zblk, flags)
    return out[:n]


# speedup vs baseline: 2.2893x; 2.2893x over previous
import functools

import jax
import jax.numpy as jnp
from jax import lax
from jax.experimental import pallas as pl
from jax.experimental.pallas import tpu as pltpu
from jax.experimental.pallas import tpu_sc as plsc

_N_ITER = 40
_NC = 2
_NS = 16
_RPT = 320
_HALF = _NS * _RPT
_NP = _NC * _HALF
_CH = 128
_D = 128


_W = 5


def _sc_step_body(v_hbm, cols_hbm, rloc_hbm, cnts_hbm, a_hbm, b_hbm, zb_hbm,
                  vout_hbm, acc_sh, gbuf, av, bv, g3, g4, idxcs, idxrs, cntv,
                  sem_i, sem_g, sem_s):
    c = lax.axis_index("c")
    s = lax.axis_index("s")
    wid = c * _NS + s
    base_l = s * _RPT
    base_g = wid * _RPT
    bufs = (gbuf, av, bv, g3, g4)

    pltpu.sync_copy(zb_hbm, gbuf)
    for blk, bsz in ((0, 128), (128, 128), (256, 64)):
        pltpu.sync_copy(gbuf.at[pl.ds(0, bsz)],
                        acc_sh.at[pl.ds(base_l + blk, bsz)])
    plsc.subcore_barrier()

    pltpu.sync_copy(cnts_hbm, cntv)
    cnt_vec = cntv[...]
    nch = jnp.where(c == 0, cnt_vec[0], cnt_vec[1])
    my_n = jnp.maximum(0, (nch - s + _NS - 1) // _NS)
    n_waves = (my_n + _W - 1) // _W

    def wave_body(w, carry):
        ts = [w * _W + b for b in range(_W)]
        valid = [t < my_n for t in ts]
        ebases = [(t * _NS + s) * _CH for t in ts]
        for b in range(_W):
            @pl.when(valid[b])
            def _issue_idx(b=b):
                pltpu.async_copy(cols_hbm.at[c, pl.ds(ebases[b], _CH)],
                                 idxcs.at[b], sem_i)
                pltpu.async_copy(rloc_hbm.at[c, pl.ds(ebases[b], _CH)],
                                 idxrs.at[b], sem_i)
        for b in range(_W):
            @pl.when(valid[b])
            def _wait_idx(b=b):
                pltpu.make_async_copy(cols_hbm.at[c, pl.ds(ebases[b], _CH)],
                                      idxcs.at[b], sem_i).wait()
                pltpu.make_async_copy(rloc_hbm.at[c, pl.ds(ebases[b], _CH)],
                                      idxrs.at[b], sem_i).wait()
        for b in range(_W):
            @pl.when(valid[b])
            def _issue_gather(b=b):
                pltpu.async_copy(
                    v_hbm.at[plsc.Indices(idxcs.at[b], ignored_value=-1)],
                    bufs[b], sem_g)
        for b in range(_W):
            @pl.when(valid[b])
            def _gather_scatter(b=b):
                pltpu.make_async_copy(
                    v_hbm.at[plsc.Indices(idxcs.at[b], ignored_value=-1)],
                    bufs[b], sem_g).wait()
                pltpu.async_copy(
                    bufs[b],
                    acc_sh.at[plsc.Indices(idxrs.at[b], ignored_value=-1)],
                    sem_s, add=True)
        for b in range(_W):
            @pl.when(valid[b])
            def _wait_scatter(b=b):
                pltpu.make_async_copy(
                    bufs[b],
                    acc_sh.at[plsc.Indices(idxrs.at[b], ignored_value=-1)],
                    sem_s).wait()
        return carry

    lax.fori_loop(0, n_waves, wave_body, 0)
    plsc.subcore_barrier()

    for blk, bsz in ((0, 128), (128, 128), (256, 64)):
        pltpu.sync_copy(acc_sh.at[pl.ds(base_l + blk, bsz)],
                        gbuf.at[pl.ds(0, bsz)])
        pltpu.sync_copy(a_hbm.at[pl.ds(base_g + blk, bsz)],
                        av.at[pl.ds(0, bsz)])
        pltpu.sync_copy(b_hbm.at[pl.ds(base_g + blk, bsz)],
                        bv.at[pl.ds(0, bsz)])

        def row_body(r, carry):
            for cc in range(_D // 16):
                sl = pl.ds(cc * 16, 16)
                av[r, sl] = av[r, sl] + bv[r, sl] * gbuf[r, sl]
            return carry

        lax.fori_loop(0, bsz, row_body, 0)
        pltpu.sync_copy(av.at[pl.ds(0, bsz)],
                        vout_hbm.at[pl.ds(base_g + blk, bsz)])


_sc_step = functools.partial(
    pl.kernel,
    out_type=jax.ShapeDtypeStruct((_NP, _D), jnp.float32),
    mesh=plsc.VectorSubcoreMesh(core_axis_name="c", subcore_axis_name="s"),
    scratch_types=[
        pltpu.VMEM_SHARED((_HALF, _D), jnp.float32),
        pltpu.VMEM((_CH, _D), jnp.float32),
        pltpu.VMEM((_CH, _D), jnp.float32),
        pltpu.VMEM((_CH, _D), jnp.float32),
        pltpu.VMEM((_CH, _D), jnp.float32),
        pltpu.VMEM((_CH, _D), jnp.float32),
        pltpu.VMEM((_W, _CH), jnp.int32),
        pltpu.VMEM((_W, _CH), jnp.int32),
        pltpu.VMEM((16,), jnp.int32),
        pltpu.SemaphoreType.DMA,
        pltpu.SemaphoreType.DMA,
        pltpu.SemaphoreType.DMA,
    ],
)(_sc_step_body)


def kernel(x, edge_index, mask):
    n, d = x.shape
    e = edge_index.shape[1]
    row = edge_index[0].astype(jnp.int32)
    col = edge_index[1].astype(jnp.int32)
    maskf = mask.astype(jnp.float32)

    ones = jnp.ones((e,), jnp.float32)
    deg = jnp.zeros((n,), jnp.float32).at[col].add(ones)
    dis = jnp.where(deg > 0, jax.lax.rsqrt(deg), 0.0)

    in0 = row < _HALF
    pos0 = jnp.cumsum(in0.astype(jnp.int32)) - 1
    n0 = pos0[-1] + 1
    pos1 = jnp.cumsum(1 - in0.astype(jnp.int32)) - 1
    pos = jnp.where(in0, pos0, e + pos1)
    cols2 = jnp.full((2 * e,), -1, jnp.int32).at[pos].set(col)
    rloc2 = jnp.full((2 * e,), -1, jnp.int32).at[pos].set(
        jnp.where(in0, row, row - _HALF))
    cols_p = cols2.reshape(2, e)
    rloc_p = rloc2.reshape(2, e)
    n1 = e - n0
    cnts = jnp.zeros((16,), jnp.int32).at[0].set(
        (n0 + _CH - 1) // _CH).at[1].set((n1 + _CH - 1) // _CH)

    xm = jnp.where(mask[:, None], x, 0.0).astype(jnp.float32)
    xm_p = jnp.zeros((_NP, d), jnp.float32).at[:n].set(xm)
    dis_p = jnp.zeros((_NP,), jnp.float32).at[:n].set(dis)
    nm_p = jnp.zeros((_NP,), jnp.float32).at[:n].set(1.0 - maskf)

    a_iter = xm_p * dis_p[:, None]
    b_iter = jnp.broadcast_to((nm_p * dis_p * dis_p)[:, None], (_NP, d))
    a_last = xm_p
    b_last = jnp.broadcast_to((nm_p * dis_p)[:, None], (_NP, d))
    zblk = jnp.zeros((_CH, _D), jnp.float32)

    v = lax.fori_loop(
        0, _N_ITER - 1,
        lambda _, vv: _sc_step(vv, cols_p, rloc_p, cnts, a_iter, b_iter, zblk),
        a_iter)
    out = _sc_step(v, cols_p, rloc_p, cnts, a_last, b_last, zblk)
    return out[:n]


# speedup vs baseline: 2.2969x; 1.0033x over previous
import functools

import jax
import jax.numpy as jnp
from jax import lax
from jax.experimental import pallas as pl
from jax.experimental.pallas import tpu as pltpu
from jax.experimental.pallas import tpu_sc as plsc

_N_ITER = 40
_NC = 2
_NS = 16
_RPT = 320
_HALF = _NS * _RPT
_NP = _NC * _HALF
_CH = 128
_D = 128


_W = 5


def _sc_step_body(v_hbm, cols_hbm, rloc_hbm, cnts_hbm, a_hbm, b_hbm, zb_hbm,
                  vout_hbm, acc_sh, gbuf, av, bv, g3, g4, idxcs, idxrs, cntv,
                  sem_i, sem_g, sem_s):
    c = lax.axis_index("c")
    s = lax.axis_index("s")
    wid = c * _NS + s
    base_l = s * _RPT
    base_g = wid * _RPT
    bufs = (gbuf, av, bv, g3, g4)

    pltpu.sync_copy(zb_hbm, gbuf)
    for blk, bsz in ((0, 128), (128, 128), (256, 64)):
        pltpu.sync_copy(gbuf.at[pl.ds(0, bsz)],
                        acc_sh.at[pl.ds(base_l + blk, bsz)])
    plsc.subcore_barrier()

    pltpu.sync_copy(cnts_hbm, cntv)
    cnt_vec = cntv[...]
    nch = jnp.where(c == 0, cnt_vec[0], cnt_vec[1])
    my_n = jnp.maximum(0, (nch - s + _NS - 1) // _NS)
    n_waves = (my_n + _W - 1) // _W

    def wave_body(w, carry):
        ts = [w * _W + b for b in range(_W)]
        valid = [t < my_n for t in ts]
        ebases = [(t * _NS + s) * _CH for t in ts]
        for b in range(_W):
            @pl.when(valid[b])
            def _issue_idx(b=b):
                pltpu.async_copy(cols_hbm.at[c, pl.ds(ebases[b], _CH)],
                                 idxcs.at[b], sem_i)
                pltpu.async_copy(rloc_hbm.at[c, pl.ds(ebases[b], _CH)],
                                 idxrs.at[b], sem_i)
        for b in range(_W):
            @pl.when(valid[b])
            def _wait_idx(b=b):
                pltpu.make_async_copy(cols_hbm.at[c, pl.ds(ebases[b], _CH)],
                                      idxcs.at[b], sem_i).wait()
                pltpu.make_async_copy(rloc_hbm.at[c, pl.ds(ebases[b], _CH)],
                                      idxrs.at[b], sem_i).wait()
        for b in range(_W):
            @pl.when(valid[b])
            def _issue_gather(b=b):
                pltpu.async_copy(
                    v_hbm.at[plsc.Indices(idxcs.at[b], ignored_value=-1)],
                    bufs[b], sem_g)
        for b in range(_W):
            @pl.when(valid[b])
            def _gather_scatter(b=b):
                pltpu.make_async_copy(
                    v_hbm.at[plsc.Indices(idxcs.at[b], ignored_value=-1)],
                    bufs[b], sem_g).wait()
                pltpu.async_copy(
                    bufs[b],
                    acc_sh.at[plsc.Indices(idxrs.at[b], ignored_value=-1)],
                    sem_s, add=True)
        for b in range(_W):
            @pl.when(valid[b])
            def _wait_scatter(b=b):
                pltpu.make_async_copy(
                    bufs[b],
                    acc_sh.at[plsc.Indices(idxrs.at[b], ignored_value=-1)],
                    sem_s).wait()
        return carry

    lax.fori_loop(0, n_waves, wave_body, 0)
    plsc.subcore_barrier()

    for blk, bsz in ((0, 128), (128, 128), (256, 64)):
        pltpu.sync_copy(acc_sh.at[pl.ds(base_l + blk, bsz)],
                        gbuf.at[pl.ds(0, bsz)])
        pltpu.sync_copy(a_hbm.at[pl.ds(base_g + blk, bsz)],
                        av.at[pl.ds(0, bsz)])
        pltpu.sync_copy(b_hbm.at[pl.ds(base_g + blk, bsz)],
                        bv.at[pl.ds(0, bsz)])

        def row_body(r, carry):
            for cc in range(_D // 16):
                sl = pl.ds(cc * 16, 16)
                av[r, sl] = av[r, sl] + bv[r, sl] * gbuf[r, sl]
            return carry

        lax.fori_loop(0, bsz, row_body, 0)
        pltpu.sync_copy(av.at[pl.ds(0, bsz)],
                        vout_hbm.at[pl.ds(base_g + blk, bsz)])


_sc_step = functools.partial(
    pl.kernel,
    out_type=jax.ShapeDtypeStruct((_NP, _D), jnp.float32),
    mesh=plsc.VectorSubcoreMesh(core_axis_name="c", subcore_axis_name="s"),
    scratch_types=[
        pltpu.VMEM_SHARED((_HALF, _D), jnp.float32),
        pltpu.VMEM((_CH, _D), jnp.float32),
        pltpu.VMEM((_CH, _D), jnp.float32),
        pltpu.VMEM((_CH, _D), jnp.float32),
        pltpu.VMEM((_CH, _D), jnp.float32),
        pltpu.VMEM((_CH, _D), jnp.float32),
        pltpu.VMEM((_W, _CH), jnp.int32),
        pltpu.VMEM((_W, _CH), jnp.int32),
        pltpu.VMEM((16,), jnp.int32),
        pltpu.SemaphoreType.DMA,
        pltpu.SemaphoreType.DMA,
        pltpu.SemaphoreType.DMA,
    ],
)(_sc_step_body)


def kernel(x, edge_index, mask):
    n, d = x.shape
    e = edge_index.shape[1]
    row = edge_index[0].astype(jnp.int32)
    col = edge_index[1].astype(jnp.int32)
    maskf = mask.astype(jnp.float32)

    ones = jnp.ones((e,), jnp.float32)
    deg = jnp.zeros((n,), jnp.float32).at[col].add(ones)
    dis = jnp.where(deg > 0, jax.lax.rsqrt(deg), 0.0)

    in0 = row < _HALF
    cs0 = jnp.cumsum(in0.astype(jnp.int32))
    n0 = cs0[-1]
    idx = jnp.arange(e, dtype=jnp.int32)
    pos = jnp.where(in0, cs0 - 1, e + idx - cs0)
    cols2 = jnp.full((2 * e,), -1, jnp.int32).at[pos].set(col)
    rloc2 = jnp.full((2 * e,), -1, jnp.int32).at[pos].set(
        jnp.where(in0, row, row - _HALF))
    cols_p = cols2.reshape(2, e)
    rloc_p = rloc2.reshape(2, e)
    n1 = e - n0
    cnts = jnp.zeros((16,), jnp.int32).at[0].set(
        (n0 + _CH - 1) // _CH).at[1].set((n1 + _CH - 1) // _CH)

    xm = jnp.where(mask[:, None], x, 0.0).astype(jnp.float32)
    xm_p = jnp.zeros((_NP, d), jnp.float32).at[:n].set(xm)
    dis_p = jnp.zeros((_NP,), jnp.float32).at[:n].set(dis)
    nm_p = jnp.zeros((_NP,), jnp.float32).at[:n].set(1.0 - maskf)

    a_iter = xm_p * dis_p[:, None]
    b_iter = jnp.broadcast_to((nm_p * dis_p * dis_p)[:, None], (_NP, d))
    a_last = xm_p
    b_last = jnp.broadcast_to((nm_p * dis_p)[:, None], (_NP, d))
    zblk = jnp.zeros((_CH, _D), jnp.float32)

    v = lax.fori_loop(
        0, _N_ITER - 1,
        lambda _, vv: _sc_step(vv, cols_p, rloc_p, cnts, a_iter, b_iter, zblk),
        a_iter)
    out = _sc_step(v, cols_p, rloc_p, cnts, a_last, b_last, zblk)
    return out[:n]
